# baseline (speedup 1.0000x reference)
import functools
import jax
import jax.numpy as jnp
from jax import lax
from jax.experimental import pallas as pl
from jax.experimental.pallas import tpu as pltpu
from jax.experimental.pallas import tpu_sc as plsc

D_IN = 2048
N_LAT = 32768
KTOP = 64
BATCH = 4096

BLK = 128
NBLK = N_LAT // BLK
NW = 32
ROWS_PER_W = BATCH // NW
NCHUNK = 4
CH = N_LAT // NCHUNK
CAND = 10496
INT_MIN = -2147483648
INT_MAX = 2147483647


def _enc_body(x_ref, w_ref, b_ref, out_ref, bm_ref):
    acc = lax.dot_general(x_ref[...], w_ref[...], (((1,), (1,)), ((), ())),
                          preferred_element_type=jnp.float32)
    acc = acc + b_ref[...]
    out_ref[...] = acc
    nb = acc.shape[1] // BLK
    bms = [jnp.max(acc[:, g * BLK:(g + 1) * BLK], axis=1, keepdims=True)
           for g in range(nb)]
    bm_ref[...] = jnp.concatenate(bms, axis=1)[None]


def _encode(x, W_enc, b_enc, BR=1024, BL=1024):
    grid = (BATCH // BR, N_LAT // BL)
    nb = BL // BLK
    pre, bm3 = pl.pallas_call(
        _enc_body,
        grid=grid,
        in_specs=[
            pl.BlockSpec((BR, D_IN), lambda r, l: (r, 0)),
            pl.BlockSpec((BL, D_IN), lambda r, l: (l, 0)),
            pl.BlockSpec((1, BL), lambda r, l: (0, l)),
        ],
        out_specs=[
            pl.BlockSpec((BR, BL), lambda r, l: (r, l)),
            pl.BlockSpec((1, BR, nb), lambda r, l: (l, r, 0)),
        ],
        out_shape=[
            jax.ShapeDtypeStruct((BATCH, N_LAT), jnp.float32),
            jax.ShapeDtypeStruct((N_LAT // BL, BATCH, nb), jnp.float32),
        ],
    )(x, W_enc, b_enc.reshape(1, N_LAT))
    bm = bm3.transpose(1, 0, 2).reshape(BATCH, NBLK)
    return pre, bm


def _dec_body(a_ref, w_ref, b_ref, out_ref):
    k = pl.program_id(1)
    a16 = a_ref[...].astype(jnp.bfloat16)
    acc = lax.dot_general(a16, w_ref[...], (((1,), (1,)), ((), ())),
                          preferred_element_type=jnp.float32)

    @pl.when(k == 0)
    def _():
        out_ref[...] = acc + b_ref[...]

    @pl.when(k > 0)
    def _():
        out_ref[...] = out_ref[...] + acc


def _decode(acts, W_dec, b_dec, BR=1024, BK=2048):
    Wd16 = W_dec.astype(jnp.bfloat16)
    grid = (BATCH // BR, N_LAT // BK)
    return pl.pallas_call(
        _dec_body,
        grid=grid,
        in_specs=[
            pl.BlockSpec((BR, BK), lambda r, k: (r, k)),
            pl.BlockSpec((D_IN, BK), lambda r, k: (0, k)),
            pl.BlockSpec((1, D_IN), lambda r, k: (0, 0)),
        ],
        out_specs=pl.BlockSpec((BR, D_IN), lambda r, k: (r, 0)),
        out_shape=jax.ShapeDtypeStruct((BATCH, D_IN), jnp.float32),
    )(acts, Wd16, b_dec.reshape(1, D_IN))


_LANE = lambda: lax.iota(jnp.int32, 16)


def _mono(b):
    return b ^ jnp.where(b < 0, jnp.int32(0x7FFFFFFF), jnp.int32(0))


def _count_ge_f(ref, nv16, thr):
    cnt = jnp.zeros((16,), jnp.int32)
    for j in range(nv16):
        v = ref[pl.ds(16 * j, 16)]
        cnt = cnt + jnp.where(v >= thr, 1, 0).astype(jnp.int32)
    return jnp.sum(cnt)


def _count_ge_k(kref, nv, thr):
    def body(j, cnt):
        v = kref[pl.ds(16 * j, 16)]
        return cnt + jnp.where(v >= thr, 1, 0).astype(jnp.int32)
    cnt = lax.fori_loop(0, nv, body, jnp.zeros((16,), jnp.int32))
    return jnp.sum(cnt)


def _bisect_t0(bm_ref):
    v0 = bm_ref[pl.ds(0, 16)]
    mx, mn = v0, v0
    for j in range(1, NBLK // 16):
        v = bm_ref[pl.ds(16 * j, 16)]
        mx = jnp.maximum(mx, v)
        mn = jnp.minimum(mn, v)
    hi = jnp.max(mx)
    lo = jnp.min(mn)

    def body(_, c):
        lo, hi = c
        mid = lo + (hi - lo) * jnp.float32(0.5)
        ge = _count_ge_f(bm_ref, NBLK // 16, mid) >= KTOP
        return (jnp.where(ge, mid, lo), jnp.where(ge, hi, mid))

    lo, hi = lax.fori_loop(0, 18, body, (lo, hi))
    return lo


def _compact64(cand_v, cand_i, cand_k, m):
    nv = (m + 15) >> 4

    def monoify(j, _):
        b = plsc.bitcast(cand_v[pl.ds(16 * j, 16)], jnp.int32)
        cand_k[pl.ds(16 * j, 16)] = _mono(b)
        return 0
    lax.fori_loop(0, nv, monoify, 0)
    sent = jnp.full((16,), INT_MIN, jnp.int32)
    cand_k[pl.ds(m, 16)] = sent

    def bis_static(_, c):
        lo, hi = c
        mid = (lo >> 1) + (hi >> 1) + (lo & hi & 1)
        cnt = jnp.zeros((16,), jnp.int32)
        for j in range(16):
            v = cand_k[pl.ds(16 * j, 16)]
            cnt = cnt + jnp.where(v >= mid, 1, 0).astype(jnp.int32)
        ge = jnp.sum(cnt) >= KTOP
        return (jnp.where(ge, mid, lo), jnp.where(ge, hi, mid))

    def bis_dyn(_, c):
        lo, hi = c
        mid = (lo >> 1) + (hi >> 1) + (lo & hi & 1)
        ge = _count_ge_k(cand_k, nv, mid) >= KTOP
        return (jnp.where(ge, mid, lo), jnp.where(ge, hi, mid))

    def run_static(_):
        def pad(j, _):
            cand_k[pl.ds(16 * j, 16)] = sent
            return 0
        lax.fori_loop((m + 16) >> 4, 16, pad, 0)
        t, _ = lax.fori_loop(0, 32, bis_static,
                             (jnp.int32(INT_MIN), jnp.int32(INT_MAX)))
        return t

    def run_dyn(_):
        t, _ = lax.fori_loop(0, 32, bis_dyn,
                             (jnp.int32(INT_MIN), jnp.int32(INT_MAX)))
        return t

    tstar = lax.cond(m <= 240, run_static, run_dyn, 0)

    def cgt(j, cnt):
        v = cand_k[pl.ds(16 * j, 16)]
        return cnt + jnp.sum(jnp.where(v > tstar, 1, 0).astype(jnp.int32))
    count_gt = lax.fori_loop(0, nv, cgt, jnp.int32(0))
    need_eq = KTOP - count_gt

    def cpart(j, carry):
        mo, ke = carry
        v = cand_k[pl.ds(16 * j, 16)]
        ci = cand_i[pl.ds(16 * j, 16)]
        gt = v > tstar
        eq = v == tstar
        eq_rank = ke + jnp.cumsum(eq.astype(jnp.int32))
        keep = gt | (eq & (eq_rank <= need_eq))
        plsc.store_compressed(cand_k.at[pl.ds(mo, 16)], v, mask=keep)
        plsc.store_compressed(cand_i.at[pl.ds(mo, 16)], ci, mask=keep)
        return (mo + jnp.sum(keep.astype(jnp.int32)),
                ke + jnp.sum(eq.astype(jnp.int32)))
    lax.fori_loop(0, nv, cpart, (jnp.int32(0), jnp.int32(0)))

    for g in range(KTOP // 16):
        k = cand_k[pl.ds(16 * g, 16)]
        cand_v[pl.ds(16 * g, 16)] = plsc.bitcast(_mono(k), jnp.float32)
    return tstar


def _build_block_lists(bm_ref, blk_l, thr):
    lane = _LANE()
    counts = []
    for c in range(NCHUNK):
        na_vec = jnp.zeros((16,), jnp.int32)
        for q in range(4):
            bmv = bm_ref[pl.ds((4 * c + q) * 16, 16)]
            amsk = bmv >= thr
            na = na_vec[0]
            plsc.store_compressed(blk_l.at[pl.ds(80 * c + na, 16)],
                                  q * 16 + lane, mask=amsk)
            na_vec = na_vec + plsc.all_reduce_population_count(amsk)
        counts.append(na_vec[0])
    return counts


def _scan_chunk(ck, c, na, blk_l, cand_v, cand_i, m_vec, thr):
    lane = _LANE()

    def blk(i, carry):
        m_vec, thr = carry
        blv = blk_l[pl.ds(80 * c + ((i >> 4) << 4), 16)]
        bid = lax.gather(
            blv, jnp.full((16, 1), i & 15, jnp.int32),
            lax.GatherDimensionNumbers(offset_dims=(),
                                       collapsed_slice_dims=(0,),
                                       start_index_map=(0,)),
            (1,), mode=lax.GatherScatterMode.PROMISE_IN_BOUNDS)
        base = bid[0] * BLK
        vs, msks, bases = [], [], []
        for j in range(8):
            v = ck[pl.ds(base + 16 * j, 16)]
            msk = v >= thr
            vs.append(v)
            msks.append(msk)
            bases.append(m_vec)
            m_vec = m_vec + plsc.all_reduce_population_count(msk)
        for j in range(8):
            m = bases[j][0]
            idx = (c * CH) + base + 16 * j + lane
            plsc.store_compressed(cand_v.at[pl.ds(m, 16)], vs[j],
                                  mask=msks[j])
            plsc.store_compressed(cand_i.at[pl.ds(m, 16)], idx,
                                  mask=msks[j])
        return (m_vec, thr)

    return lax.fori_loop(0, na, blk, (m_vec, thr))


def _maybe_compact(cand_v, cand_i, cand_k, m_vec, thr):
    m = m_vec[0]

    def do_compact(c2):
        m_vec2, thr2 = c2
        tstar = _compact64(cand_v, cand_i, cand_k, m_vec2[0])
        tv = plsc.bitcast(_mono(jnp.full((16,), tstar, jnp.int32)),
                          jnp.float32)
        return (jnp.full((16,), KTOP, jnp.int32), jnp.max(tv))

    return lax.cond(m > CAND - CH - 16, do_compact, lambda c2: c2,
                    (m_vec, thr))


def _extract64(cand_k, cand_i, outv, outi):
    lane = _LANE()

    def step(k, carry):
        wk, wi = carry
        vmax = jnp.full((16,), INT_MIN, jnp.int32)
        vcomb = jnp.full((16,), INT_MAX, jnp.int32)
        for j in range(KTOP // 16):
            cv = cand_k[pl.ds(16 * j, 16)]
            ci = cand_i[pl.ds(16 * j, 16)]
            comb = (ci << 8) | (16 * j + lane)
            better = (cv > vmax) | ((cv == vmax) & (comb < vcomb))
            vmax = jnp.where(better, cv, vmax)
            vcomb = jnp.where(better, comb, vcomb)
        mk = jnp.max(vmax)
        wc = jnp.min(jnp.where(vmax == mk, vcomb, INT_MAX))
        widx = wc >> 8
        wpos = wc & 255
        kmod = k & 15
        wk = jnp.where(lane == kmod, mk, wk)
        wi = jnp.where(lane == kmod, widx, wi)

        @pl.when(kmod == 15)
        def _():
            grp = k >> 4
            outv[pl.ds(16 * grp, 16)] = plsc.bitcast(_mono(wk), jnp.float32)
            outi[pl.ds(16 * grp, 16)] = wi

        wsrc = wpos >> 4
        cv = cand_k[pl.ds(16 * wsrc, 16)]
        cand_k[pl.ds(16 * wsrc, 16)] = jnp.where(
            lane == (wpos & 15), INT_MIN, cv)
        return (wk, wi)

    lax.fori_loop(0, KTOP, step,
                  (jnp.zeros((16,), jnp.int32), jnp.zeros((16,), jnp.int32)))


def _sc_topk(pre, bm):
    mesh = plsc.VectorSubcoreMesh(core_axis_name="c", subcore_axis_name="s")
    zeros16 = lambda: jnp.zeros((16,), jnp.float32)

    @functools.partial(
        pl.kernel,
        out_type=[jax.ShapeDtypeStruct((BATCH, N_LAT), jnp.float32),
                  jax.ShapeDtypeStruct((BATCH, KTOP), jnp.int32)],
        mesh=mesh,
        compiler_params=pltpu.CompilerParams(needs_layout_passes=False),
        scratch_types=[
            pltpu.VMEM((CH,), jnp.float32),
            pltpu.VMEM((CH,), jnp.float32),
            pltpu.VMEM((NBLK,), jnp.float32),
            pltpu.VMEM((NBLK,), jnp.float32),
            pltpu.VMEM((CAND + 16,), jnp.float32),
            pltpu.VMEM((CAND + 16,), jnp.int32),
            pltpu.VMEM((CAND + 16,), jnp.int32),
            pltpu.VMEM((336,), jnp.int32),
            pltpu.VMEM((N_LAT,), jnp.float32),
            pltpu.VMEM((N_LAT,), jnp.float32),
            pltpu.VMEM((KTOP,), jnp.float32),
            pltpu.VMEM((KTOP,), jnp.float32),
            pltpu.VMEM((KTOP,), jnp.int32),
            pltpu.VMEM((KTOP,), jnp.int32),
            pltpu.SemaphoreType.DMA,
            pltpu.SemaphoreType.DMA,
            pltpu.SemaphoreType.DMA,
            pltpu.SemaphoreType.DMA,
            pltpu.SemaphoreType.DMA,
            pltpu.SemaphoreType.DMA,
        ],
    )
    def k(pre_hbm, bm_hbm, acts_hbm, idx_hbm,
          ck_a, ck_b, bm_a, bm_b, cand_v, cand_i, cand_k, blk_l,
          acts_a, acts_b, outv_a, outv_b, outi_a, outi_b,
          sem_ca, sem_cb, sem_bma, sem_bmb, sem_oa, sem_ob):
        wid = lax.axis_index("s") * 2 + lax.axis_index("c")
        base = wid * ROWS_PER_W

        def z(j, _):
            acts_a[pl.ds(16 * j, 16)] = zeros16()
            acts_b[pl.ds(16 * j, 16)] = zeros16()
            return 0
        lax.fori_loop(0, N_LAT // 16, z, 0)

        pltpu.async_copy(bm_hbm.at[base], bm_a, sem_bma)

        def do_row(i, bm_mine, sem_bm_mine, bm_next, sem_bm_next,
                   acts_buf, outv, outi, sem_o):
            r = base + i
            nxt = base + jnp.minimum(i + 1, ROWS_PER_W - 1)
            pltpu.async_copy(bm_hbm.at[nxt], bm_next, sem_bm_next)
            h0 = pltpu.async_copy(pre_hbm.at[r, pl.ds(0, CH)], ck_a, sem_ca)
            h1 = pltpu.async_copy(pre_hbm.at[r, pl.ds(CH, CH)], ck_b, sem_cb)
            pltpu.make_async_copy(bm_hbm.at[r], bm_mine, sem_bm_mine).wait()
            thr = _bisect_t0(bm_mine)
            na = _build_block_lists(bm_mine, blk_l, thr)

            @pl.when(i >= 2)
            def _():
                pltpu.make_async_copy(acts_buf, acts_hbm.at[r], sem_o).wait()
                pltpu.make_async_copy(outi, idx_hbm.at[r], sem_o).wait()
                for g in range(KTOP // 16):
                    iv = outi[pl.ds(16 * g, 16)]
                    plsc.store_scatter(acts_buf, [iv], zeros16())

            mv = jnp.zeros((16,), jnp.int32)
            h0.wait()
            mv, thr = _scan_chunk(ck_a, 0, na[0], blk_l, cand_v, cand_i,
                                  mv, thr)
            h2 = pltpu.async_copy(pre_hbm.at[r, pl.ds(2 * CH, CH)], ck_a,
                                  sem_ca)
            h1.wait()
            mv, thr = _maybe_compact(cand_v, cand_i, cand_k, mv, thr)
            mv, thr = _scan_chunk(ck_b, 1, na[1], blk_l, cand_v, cand_i,
                                  mv, thr)
            h3 = pltpu.async_copy(pre_hbm.at[r, pl.ds(3 * CH, CH)], ck_b,
                                  sem_cb)
            h2.wait()
            mv, thr = _maybe_compact(cand_v, cand_i, cand_k, mv, thr)
            mv, thr = _scan_chunk(ck_a, 2, na[2], blk_l, cand_v, cand_i,
                                  mv, thr)
            h3.wait()
            mv, thr = _maybe_compact(cand_v, cand_i, cand_k, mv, thr)
            mv, thr = _scan_chunk(ck_b, 3, na[3], blk_l, cand_v, cand_i,
                                  mv, thr)

            _compact64(cand_v, cand_i, cand_k, mv[0])
            _extract64(cand_k, cand_i, outv, outi)

            for g in range(KTOP // 16):
                iv = outi[pl.ds(16 * g, 16)]
                vv = jnp.maximum(outv[pl.ds(16 * g, 16)], jnp.float32(0.0))
                plsc.store_scatter(acts_buf, [iv], vv)
            pltpu.async_copy(acts_buf, acts_hbm.at[r], sem_o)
            pltpu.async_copy(outi, idx_hbm.at[r], sem_o)

        def pair(p, _):
            do_row(2 * p, bm_a, sem_bma, bm_b, sem_bmb,
                   acts_a, outv_a, outi_a, sem_oa)
            do_row(2 * p + 1, bm_b, sem_bmb, bm_a, sem_bma,
                   acts_b, outv_b, outi_b, sem_ob)
            return 0
        lax.fori_loop(0, ROWS_PER_W // 2, pair, 0)

        pltpu.make_async_copy(bm_hbm.at[base], bm_a, sem_bma).wait()
        pltpu.make_async_copy(acts_a, acts_hbm.at[base], sem_oa).wait()
        pltpu.make_async_copy(outi_a, idx_hbm.at[base], sem_oa).wait()
        pltpu.make_async_copy(acts_b, acts_hbm.at[base], sem_ob).wait()
        pltpu.make_async_copy(outi_b, idx_hbm.at[base], sem_ob).wait()

    return k(pre, bm)


def kernel(x, W_enc, b_enc, W_dec, b_dec):
    pre_acts, bm = _encode(x, W_enc, b_enc)
    acts, topk_idx = _sc_topk(pre_acts, bm)
    recon = _decode(acts, W_dec, b_dec)
    return (recon, acts, topk_idx)


# speedup vs baseline: 1.0577x; 1.0577x over previous
import functools
import jax
import jax.numpy as jnp
from jax import lax
from jax.experimental import pallas as pl
from jax.experimental.pallas import tpu as pltpu
from jax.experimental.pallas import tpu_sc as plsc

D_IN = 2048
N_LAT = 32768
KTOP = 64
BATCH = 4096

BLK = 128
NBLK = N_LAT // BLK
NW = 32
ROWS_PER_W = BATCH // NW
NCHUNK = 4
CH = N_LAT // NCHUNK
CAND = 10496
INT_MIN = -2147483648
INT_MAX = 2147483647


def _enc_body(x_ref, w_ref, b_ref, out_ref, bm_ref):
    acc = lax.dot_general(x_ref[...], w_ref[...], (((1,), (1,)), ((), ())),
                          preferred_element_type=jnp.float32)
    acc = acc + b_ref[...]
    out_ref[...] = acc
    nb = acc.shape[1] // BLK
    bms = [jnp.max(acc[:, g * BLK:(g + 1) * BLK], axis=1, keepdims=True)
           for g in range(nb)]
    bm_ref[...] = jnp.concatenate(bms, axis=1)[None]


def _encode(x, W_enc, b_enc, nrows, BR=1024, BL=1024):
    grid = (nrows // BR, N_LAT // BL)
    nb = BL // BLK
    pre, bm3 = pl.pallas_call(
        _enc_body,
        grid=grid,
        in_specs=[
            pl.BlockSpec((BR, D_IN), lambda r, l: (r, 0)),
            pl.BlockSpec((BL, D_IN), lambda r, l: (l, 0)),
            pl.BlockSpec((1, BL), lambda r, l: (0, l)),
        ],
        out_specs=[
            pl.BlockSpec((BR, BL), lambda r, l: (r, l)),
            pl.BlockSpec((1, BR, nb), lambda r, l: (l, r, 0)),
        ],
        out_shape=[
            jax.ShapeDtypeStruct((nrows, N_LAT), jnp.float32),
            jax.ShapeDtypeStruct((N_LAT // BL, nrows, nb), jnp.float32),
        ],
    )(x, W_enc, b_enc.reshape(1, N_LAT))
    bm = bm3.transpose(1, 0, 2).reshape(nrows, NBLK)
    return pre, bm


def _dec_body(a_ref, w_ref, b_ref, out_ref):
    k = pl.program_id(1)
    a16 = a_ref[...].astype(jnp.bfloat16)
    acc = lax.dot_general(a16, w_ref[...], (((1,), (1,)), ((), ())),
                          preferred_element_type=jnp.float32)

    @pl.when(k == 0)
    def _():
        out_ref[...] = acc + b_ref[...]

    @pl.when(k > 0)
    def _():
        out_ref[...] = out_ref[...] + acc


def _decode(acts, Wd16, b_dec, nrows, BR=1024, BK=2048):
    grid = (nrows // BR, N_LAT // BK)
    return pl.pallas_call(
        _dec_body,
        grid=grid,
        in_specs=[
            pl.BlockSpec((BR, BK), lambda r, k: (r, k)),
            pl.BlockSpec((D_IN, BK), lambda r, k: (0, k)),
            pl.BlockSpec((1, D_IN), lambda r, k: (0, 0)),
        ],
        out_specs=pl.BlockSpec((BR, D_IN), lambda r, k: (r, 0)),
        out_shape=jax.ShapeDtypeStruct((nrows, D_IN), jnp.float32),
    )(acts, Wd16, b_dec.reshape(1, D_IN))


_LANE = lambda: lax.iota(jnp.int32, 16)


def _mono(b):
    return b ^ jnp.where(b < 0, jnp.int32(0x7FFFFFFF), jnp.int32(0))


def _count_ge_f(ref, nv16, thr):
    cnt = jnp.zeros((16,), jnp.int32)
    for j in range(nv16):
        v = ref[pl.ds(16 * j, 16)]
        cnt = cnt + jnp.where(v >= thr, 1, 0).astype(jnp.int32)
    return jnp.sum(cnt)


def _count_ge_k(kref, nv, thr):
    def body(j, cnt):
        v = kref[pl.ds(16 * j, 16)]
        return cnt + jnp.where(v >= thr, 1, 0).astype(jnp.int32)
    cnt = lax.fori_loop(0, nv, body, jnp.zeros((16,), jnp.int32))
    return jnp.sum(cnt)


def _bisect_t0(bm_ref):
    v0 = bm_ref[pl.ds(0, 16)]
    mx, mn = v0, v0
    for j in range(1, NBLK // 16):
        v = bm_ref[pl.ds(16 * j, 16)]
        mx = jnp.maximum(mx, v)
        mn = jnp.minimum(mn, v)
    hi = jnp.max(mx)
    lo = jnp.min(mn)

    def body(_, c):
        lo, hi = c
        mid = lo + (hi - lo) * jnp.float32(0.5)
        ge = _count_ge_f(bm_ref, NBLK // 16, mid) >= KTOP
        return (jnp.where(ge, mid, lo), jnp.where(ge, hi, mid))

    lo, hi = lax.fori_loop(0, 18, body, (lo, hi))
    return lo


def _compact64(cand_v, cand_i, cand_k, m):
    nv = (m + 15) >> 4

    def monoify(j, _):
        b = plsc.bitcast(cand_v[pl.ds(16 * j, 16)], jnp.int32)
        cand_k[pl.ds(16 * j, 16)] = _mono(b)
        return 0
    lax.fori_loop(0, nv, monoify, 0)
    sent = jnp.full((16,), INT_MIN, jnp.int32)
    cand_k[pl.ds(m, 16)] = sent

    def bis_static(_, c):
        lo, hi = c
        mid = (lo >> 1) + (hi >> 1) + (lo & hi & 1)
        cnt = jnp.zeros((16,), jnp.int32)
        for j in range(16):
            v = cand_k[pl.ds(16 * j, 16)]
            cnt = cnt + jnp.where(v >= mid, 1, 0).astype(jnp.int32)
        ge = jnp.sum(cnt) >= KTOP
        return (jnp.where(ge, mid, lo), jnp.where(ge, hi, mid))

    def bis_dyn(_, c):
        lo, hi = c
        mid = (lo >> 1) + (hi >> 1) + (lo & hi & 1)
        ge = _count_ge_k(cand_k, nv, mid) >= KTOP
        return (jnp.where(ge, mid, lo), jnp.where(ge, hi, mid))

    def run_static(_):
        def pad(j, _):
            cand_k[pl.ds(16 * j, 16)] = sent
            return 0
        lax.fori_loop((m + 16) >> 4, 16, pad, 0)
        t, _ = lax.fori_loop(0, 32, bis_static,
                             (jnp.int32(INT_MIN), jnp.int32(INT_MAX)))
        return t

    def run_dyn(_):
        t, _ = lax.fori_loop(0, 32, bis_dyn,
                             (jnp.int32(INT_MIN), jnp.int32(INT_MAX)))
        return t

    tstar = lax.cond(m <= 240, run_static, run_dyn, 0)

    def cgt(j, cnt):
        v = cand_k[pl.ds(16 * j, 16)]
        return cnt + jnp.sum(jnp.where(v > tstar, 1, 0).astype(jnp.int32))
    count_gt = lax.fori_loop(0, nv, cgt, jnp.int32(0))
    need_eq = KTOP - count_gt

    def cpart(j, carry):
        mo, ke = carry
        v = cand_k[pl.ds(16 * j, 16)]
        ci = cand_i[pl.ds(16 * j, 16)]
        gt = v > tstar
        eq = v == tstar
        eq_rank = ke + jnp.cumsum(eq.astype(jnp.int32))
        keep = gt | (eq & (eq_rank <= need_eq))
        plsc.store_compressed(cand_k.at[pl.ds(mo, 16)], v, mask=keep)
        plsc.store_compressed(cand_i.at[pl.ds(mo, 16)], ci, mask=keep)
        return (mo + jnp.sum(keep.astype(jnp.int32)),
                ke + jnp.sum(eq.astype(jnp.int32)))
    lax.fori_loop(0, nv, cpart, (jnp.int32(0), jnp.int32(0)))

    for g in range(KTOP // 16):
        k = cand_k[pl.ds(16 * g, 16)]
        cand_v[pl.ds(16 * g, 16)] = plsc.bitcast(_mono(k), jnp.float32)
    return tstar


def _build_block_lists(bm_ref, blk_l, thr):
    lane = _LANE()
    counts = []
    for c in range(NCHUNK):
        na_vec = jnp.zeros((16,), jnp.int32)
        for q in range(4):
            bmv = bm_ref[pl.ds((4 * c + q) * 16, 16)]
            amsk = bmv >= thr
            na = na_vec[0]
            plsc.store_compressed(blk_l.at[pl.ds(80 * c + na, 16)],
                                  q * 16 + lane, mask=amsk)
            na_vec = na_vec + plsc.all_reduce_population_count(amsk)
        counts.append(na_vec[0])
    return counts


def _scan_chunk(ck, c, na, blk_l, cand_v, cand_i, m_vec, thr):
    lane = _LANE()

    def blk(i, carry):
        m_vec, thr = carry
        blv = blk_l[pl.ds(80 * c + ((i >> 4) << 4), 16)]
        bid = lax.gather(
            blv, jnp.full((16, 1), i & 15, jnp.int32),
            lax.GatherDimensionNumbers(offset_dims=(),
                                       collapsed_slice_dims=(0,),
                                       start_index_map=(0,)),
            (1,), mode=lax.GatherScatterMode.PROMISE_IN_BOUNDS)
        base = bid[0] * BLK
        vs, msks, bases = [], [], []
        for j in range(8):
            v = ck[pl.ds(base + 16 * j, 16)]
            msk = v >= thr
            vs.append(v)
            msks.append(msk)
            bases.append(m_vec)
            m_vec = m_vec + plsc.all_reduce_population_count(msk)
        for j in range(8):
            m = bases[j][0]
            idx = (c * CH) + base + 16 * j + lane
            plsc.store_compressed(cand_v.at[pl.ds(m, 16)], vs[j],
                                  mask=msks[j])
            plsc.store_compressed(cand_i.at[pl.ds(m, 16)], idx,
                                  mask=msks[j])
        return (m_vec, thr)

    return lax.fori_loop(0, na, blk, (m_vec, thr))


def _maybe_compact(cand_v, cand_i, cand_k, m_vec, thr):
    m = m_vec[0]

    def do_compact(c2):
        m_vec2, thr2 = c2
        tstar = _compact64(cand_v, cand_i, cand_k, m_vec2[0])
        tv = plsc.bitcast(_mono(jnp.full((16,), tstar, jnp.int32)),
                          jnp.float32)
        return (jnp.full((16,), KTOP, jnp.int32), jnp.max(tv))

    return lax.cond(m > CAND - CH - 16, do_compact, lambda c2: c2,
                    (m_vec, thr))


def _extract64(cand_k, cand_i, outv, outi):
    lane = _LANE()

    def step(k, carry):
        wk, wi = carry
        vmax = jnp.full((16,), INT_MIN, jnp.int32)
        vcomb = jnp.full((16,), INT_MAX, jnp.int32)
        for j in range(KTOP // 16):
            cv = cand_k[pl.ds(16 * j, 16)]
            ci = cand_i[pl.ds(16 * j, 16)]
            comb = (ci << 8) | (16 * j + lane)
            better = (cv > vmax) | ((cv == vmax) & (comb < vcomb))
            vmax = jnp.where(better, cv, vmax)
            vcomb = jnp.where(better, comb, vcomb)
        mk = jnp.max(vmax)
        wc = jnp.min(jnp.where(vmax == mk, vcomb, INT_MAX))
        widx = wc >> 8
        wpos = wc & 255
        kmod = k & 15
        wk = jnp.where(lane == kmod, mk, wk)
        wi = jnp.where(lane == kmod, widx, wi)

        @pl.when(kmod == 15)
        def _():
            grp = k >> 4
            outv[pl.ds(16 * grp, 16)] = plsc.bitcast(_mono(wk), jnp.float32)
            outi[pl.ds(16 * grp, 16)] = wi

        wsrc = wpos >> 4
        cv = cand_k[pl.ds(16 * wsrc, 16)]
        cand_k[pl.ds(16 * wsrc, 16)] = jnp.where(
            lane == (wpos & 15), INT_MIN, cv)
        return (wk, wi)

    lax.fori_loop(0, KTOP, step,
                  (jnp.zeros((16,), jnp.int32), jnp.zeros((16,), jnp.int32)))


def _sc_topk(pre, bm, nrows):
    mesh = plsc.VectorSubcoreMesh(core_axis_name="c", subcore_axis_name="s")
    zeros16 = lambda: jnp.zeros((16,), jnp.float32)
    rpw = nrows // NW

    @functools.partial(
        pl.kernel,
        out_type=[jax.ShapeDtypeStruct((nrows, N_LAT), jnp.float32),
                  jax.ShapeDtypeStruct((nrows, KTOP), jnp.int32)],
        mesh=mesh,
        compiler_params=pltpu.CompilerParams(needs_layout_passes=False),
        scratch_types=[
            pltpu.VMEM((CH,), jnp.float32),
            pltpu.VMEM((CH,), jnp.float32),
            pltpu.VMEM((NBLK,), jnp.float32),
            pltpu.VMEM((NBLK,), jnp.float32),
            pltpu.VMEM((CAND + 16,), jnp.float32),
            pltpu.VMEM((CAND + 16,), jnp.int32),
            pltpu.VMEM((CAND + 16,), jnp.int32),
            pltpu.VMEM((336,), jnp.int32),
            pltpu.VMEM((N_LAT,), jnp.float32),
            pltpu.VMEM((N_LAT,), jnp.float32),
            pltpu.VMEM((KTOP,), jnp.float32),
            pltpu.VMEM((KTOP,), jnp.float32),
            pltpu.VMEM((KTOP,), jnp.int32),
            pltpu.VMEM((KTOP,), jnp.int32),
            pltpu.SemaphoreType.DMA,
            pltpu.SemaphoreType.DMA,
            pltpu.SemaphoreType.DMA,
            pltpu.SemaphoreType.DMA,
            pltpu.SemaphoreType.DMA,
            pltpu.SemaphoreType.DMA,
        ],
    )
    def k(pre_hbm, bm_hbm, acts_hbm, idx_hbm,
          ck_a, ck_b, bm_a, bm_b, cand_v, cand_i, cand_k, blk_l,
          acts_a, acts_b, outv_a, outv_b, outi_a, outi_b,
          sem_ca, sem_cb, sem_bma, sem_bmb, sem_oa, sem_ob):
        wid = lax.axis_index("s") * 2 + lax.axis_index("c")
        base = wid * rpw

        def z(j, _):
            acts_a[pl.ds(16 * j, 16)] = zeros16()
            acts_b[pl.ds(16 * j, 16)] = zeros16()
            return 0
        lax.fori_loop(0, N_LAT // 16, z, 0)

        pltpu.async_copy(bm_hbm.at[base], bm_a, sem_bma)

        def do_row(i, bm_mine, sem_bm_mine, bm_next, sem_bm_next,
                   acts_buf, outv, outi, sem_o):
            r = base + i
            nxt = base + jnp.minimum(i + 1, rpw - 1)
            pltpu.async_copy(bm_hbm.at[nxt], bm_next, sem_bm_next)
            h0 = pltpu.async_copy(pre_hbm.at[r, pl.ds(0, CH)], ck_a, sem_ca)
            h1 = pltpu.async_copy(pre_hbm.at[r, pl.ds(CH, CH)], ck_b, sem_cb)
            pltpu.make_async_copy(bm_hbm.at[r], bm_mine, sem_bm_mine).wait()
            thr = _bisect_t0(bm_mine)
            na = _build_block_lists(bm_mine, blk_l, thr)

            @pl.when(i >= 2)
            def _():
                pltpu.make_async_copy(acts_buf, acts_hbm.at[r], sem_o).wait()
                pltpu.make_async_copy(outi, idx_hbm.at[r], sem_o).wait()
                for g in range(KTOP // 16):
                    iv = outi[pl.ds(16 * g, 16)]
                    plsc.store_scatter(acts_buf, [iv], zeros16())

            mv = jnp.zeros((16,), jnp.int32)
            h0.wait()
            mv, thr = _scan_chunk(ck_a, 0, na[0], blk_l, cand_v, cand_i,
                                  mv, thr)
            h2 = pltpu.async_copy(pre_hbm.at[r, pl.ds(2 * CH, CH)], ck_a,
                                  sem_ca)
            h1.wait()
            mv, thr = _maybe_compact(cand_v, cand_i, cand_k, mv, thr)
            mv, thr = _scan_chunk(ck_b, 1, na[1], blk_l, cand_v, cand_i,
                                  mv, thr)
            h3 = pltpu.async_copy(pre_hbm.at[r, pl.ds(3 * CH, CH)], ck_b,
                                  sem_cb)
            h2.wait()
            mv, thr = _maybe_compact(cand_v, cand_i, cand_k, mv, thr)
            mv, thr = _scan_chunk(ck_a, 2, na[2], blk_l, cand_v, cand_i,
                                  mv, thr)
            h3.wait()
            mv, thr = _maybe_compact(cand_v, cand_i, cand_k, mv, thr)
            mv, thr = _scan_chunk(ck_b, 3, na[3], blk_l, cand_v, cand_i,
                                  mv, thr)

            _compact64(cand_v, cand_i, cand_k, mv[0])
            _extract64(cand_k, cand_i, outv, outi)

            for g in range(KTOP // 16):
                iv = outi[pl.ds(16 * g, 16)]
                vv = jnp.maximum(outv[pl.ds(16 * g, 16)], jnp.float32(0.0))
                plsc.store_scatter(acts_buf, [iv], vv)
            pltpu.async_copy(acts_buf, acts_hbm.at[r], sem_o)
            pltpu.async_copy(outi, idx_hbm.at[r], sem_o)

        def pair(p, _):
            do_row(2 * p, bm_a, sem_bma, bm_b, sem_bmb,
                   acts_a, outv_a, outi_a, sem_oa)
            do_row(2 * p + 1, bm_b, sem_bmb, bm_a, sem_bma,
                   acts_b, outv_b, outi_b, sem_ob)
            return 0
        lax.fori_loop(0, rpw // 2, pair, 0)

        pltpu.make_async_copy(bm_hbm.at[base], bm_a, sem_bma).wait()
        pltpu.make_async_copy(acts_a, acts_hbm.at[base], sem_oa).wait()
        pltpu.make_async_copy(outi_a, idx_hbm.at[base], sem_oa).wait()
        pltpu.make_async_copy(acts_b, acts_hbm.at[base], sem_ob).wait()
        pltpu.make_async_copy(outi_b, idx_hbm.at[base], sem_ob).wait()

    return k(pre, bm)


def kernel(x, W_enc, b_enc, W_dec, b_dec):
    Wd16 = W_dec.astype(jnp.bfloat16)
    H = BATCH // 2
    recons, actss, idxs = [], [], []
    for h in range(2):
        xh = lax.slice_in_dim(x, h * H, (h + 1) * H, axis=0)
        pre, bm = _encode(xh, W_enc, b_enc, H)
        acts_h, idx_h = _sc_topk(pre, bm, H)
        recons.append(_decode(acts_h, Wd16, b_dec, H))
        actss.append(acts_h)
        idxs.append(idx_h)
    recon = jnp.concatenate(recons, axis=0)
    acts = jnp.concatenate(actss, axis=0)
    topk_idx = jnp.concatenate(idxs, axis=0)
    return (recon, acts, topk_idx)


# speedup vs baseline: 1.1570x; 1.0939x over previous
import functools
import jax
import jax.numpy as jnp
from jax import lax
from jax.experimental import pallas as pl
from jax.experimental.pallas import tpu as pltpu
from jax.experimental.pallas import tpu_sc as plsc

D_IN = 2048
N_LAT = 32768
KTOP = 64
BATCH = 4096

BLK = 128
NBLK = N_LAT // BLK
NW = 32
ROWS_PER_W = BATCH // NW
NCHUNK = 4
CH = N_LAT // NCHUNK
CAND = 10496
INT_MIN = -2147483648
INT_MAX = 2147483647


def _enc_body(x_ref, w_ref, b_ref, out_ref, bm_ref):
    acc = lax.dot_general(x_ref[...], w_ref[...], (((1,), (1,)), ((), ())),
                          preferred_element_type=jnp.float32)
    acc = acc + b_ref[...]
    out_ref[...] = acc
    nb = acc.shape[1] // BLK
    bms = [jnp.max(acc[:, g * BLK:(g + 1) * BLK], axis=1, keepdims=True)
           for g in range(nb)]
    bm_ref[...] = jnp.concatenate(bms, axis=1)[None]


def _encode(x, W_enc, b_enc, nrows, BR=1024, BL=1024):
    grid = (nrows // BR, N_LAT // BL)
    nb = BL // BLK
    pre, bm3 = pl.pallas_call(
        _enc_body,
        grid=grid,
        in_specs=[
            pl.BlockSpec((BR, D_IN), lambda r, l: (r, 0)),
            pl.BlockSpec((BL, D_IN), lambda r, l: (l, 0)),
            pl.BlockSpec((1, BL), lambda r, l: (0, l)),
        ],
        out_specs=[
            pl.BlockSpec((BR, BL), lambda r, l: (r, l)),
            pl.BlockSpec((1, BR, nb), lambda r, l: (l, r, 0)),
        ],
        out_shape=[
            jax.ShapeDtypeStruct((nrows, N_LAT), jnp.float32),
            jax.ShapeDtypeStruct((N_LAT // BL, nrows, nb), jnp.float32),
        ],
    )(x, W_enc, b_enc.reshape(1, N_LAT))
    bm = bm3.transpose(1, 0, 2).reshape(nrows, NBLK)
    return pre, bm


def _dec_body(a_ref, w_ref, b_ref, out_ref):
    k = pl.program_id(1)
    a16 = a_ref[...].astype(jnp.bfloat16)
    acc = lax.dot_general(a16, w_ref[...], (((1,), (1,)), ((), ())),
                          preferred_element_type=jnp.float32)

    @pl.when(k == 0)
    def _():
        out_ref[...] = acc + b_ref[...]

    @pl.when(k > 0)
    def _():
        out_ref[...] = out_ref[...] + acc


def _decode(acts, Wd16, b_dec, nrows, BR=1024, BK=2048):
    grid = (nrows // BR, N_LAT // BK)
    return pl.pallas_call(
        _dec_body,
        grid=grid,
        in_specs=[
            pl.BlockSpec((BR, BK), lambda r, k: (r, k)),
            pl.BlockSpec((D_IN, BK), lambda r, k: (0, k)),
            pl.BlockSpec((1, D_IN), lambda r, k: (0, 0)),
        ],
        out_specs=pl.BlockSpec((BR, D_IN), lambda r, k: (r, 0)),
        out_shape=jax.ShapeDtypeStruct((nrows, D_IN), jnp.float32),
    )(acts, Wd16, b_dec.reshape(1, D_IN))


_LANE = lambda: lax.iota(jnp.int32, 16)


def _mono(b):
    return b ^ jnp.where(b < 0, jnp.int32(0x7FFFFFFF), jnp.int32(0))


def _count_ge_f(ref, nv16, thr):
    cnt = jnp.zeros((16,), jnp.int32)
    for j in range(nv16):
        v = ref[pl.ds(16 * j, 16)]
        cnt = cnt + jnp.where(v >= thr, 1, 0).astype(jnp.int32)
    return jnp.sum(cnt)


def _count_ge_k(kref, nv, thr):
    def body(j, cnt):
        v = kref[pl.ds(16 * j, 16)]
        return cnt + jnp.where(v >= thr, 1, 0).astype(jnp.int32)
    cnt = lax.fori_loop(0, nv, body, jnp.zeros((16,), jnp.int32))
    return jnp.sum(cnt)


def _bisect_t0(bm_ref):
    v0 = bm_ref[pl.ds(0, 16)]
    mx, mn = v0, v0
    for j in range(1, NBLK // 16):
        v = bm_ref[pl.ds(16 * j, 16)]
        mx = jnp.maximum(mx, v)
        mn = jnp.minimum(mn, v)
    hi = jnp.max(mx)
    lo = jnp.min(mn)

    def body(_, c):
        lo, hi = c
        mid = lo + (hi - lo) * jnp.float32(0.5)
        ge = _count_ge_f(bm_ref, NBLK // 16, mid) >= KTOP
        return (jnp.where(ge, mid, lo), jnp.where(ge, hi, mid))

    lo, hi = lax.fori_loop(0, 18, body, (lo, hi))
    return lo


def _compact64(cand_v, cand_i, cand_k, m):
    nv = (m + 15) >> 4

    def monoify(j, _):
        b = plsc.bitcast(cand_v[pl.ds(16 * j, 16)], jnp.int32)
        cand_k[pl.ds(16 * j, 16)] = _mono(b)
        return 0
    lax.fori_loop(0, nv, monoify, 0)
    sent = jnp.full((16,), INT_MIN, jnp.int32)
    cand_k[pl.ds(m, 16)] = sent

    def bis_static(_, c):
        lo, hi = c
        mid = (lo >> 1) + (hi >> 1) + (lo & hi & 1)
        cnt = jnp.zeros((16,), jnp.int32)
        for j in range(16):
            v = cand_k[pl.ds(16 * j, 16)]
            cnt = cnt + jnp.where(v >= mid, 1, 0).astype(jnp.int32)
        ge = jnp.sum(cnt) >= KTOP
        return (jnp.where(ge, mid, lo), jnp.where(ge, hi, mid))

    def bis_dyn(_, c):
        lo, hi = c
        mid = (lo >> 1) + (hi >> 1) + (lo & hi & 1)
        ge = _count_ge_k(cand_k, nv, mid) >= KTOP
        return (jnp.where(ge, mid, lo), jnp.where(ge, hi, mid))

    def run_static(_):
        def pad(j, _):
            cand_k[pl.ds(16 * j, 16)] = sent
            return 0
        lax.fori_loop((m + 16) >> 4, 16, pad, 0)
        t, _ = lax.fori_loop(0, 32, bis_static,
                             (jnp.int32(INT_MIN), jnp.int32(INT_MAX)))
        return t

    def run_dyn(_):
        t, _ = lax.fori_loop(0, 32, bis_dyn,
                             (jnp.int32(INT_MIN), jnp.int32(INT_MAX)))
        return t

    tstar = lax.cond(m <= 240, run_static, run_dyn, 0)

    def cgt(j, cnt):
        v = cand_k[pl.ds(16 * j, 16)]
        return cnt + jnp.sum(jnp.where(v > tstar, 1, 0).astype(jnp.int32))
    count_gt = lax.fori_loop(0, nv, cgt, jnp.int32(0))
    need_eq = KTOP - count_gt

    def cpart(j, carry):
        mo, ke = carry
        v = cand_k[pl.ds(16 * j, 16)]
        ci = cand_i[pl.ds(16 * j, 16)]
        gt = v > tstar
        eq = v == tstar
        eq_rank = ke + jnp.cumsum(eq.astype(jnp.int32))
        keep = gt | (eq & (eq_rank <= need_eq))
        plsc.store_compressed(cand_k.at[pl.ds(mo, 16)], v, mask=keep)
        plsc.store_compressed(cand_i.at[pl.ds(mo, 16)], ci, mask=keep)
        return (mo + jnp.sum(keep.astype(jnp.int32)),
                ke + jnp.sum(eq.astype(jnp.int32)))
    lax.fori_loop(0, nv, cpart, (jnp.int32(0), jnp.int32(0)))

    for g in range(KTOP // 16):
        k = cand_k[pl.ds(16 * g, 16)]
        cand_v[pl.ds(16 * g, 16)] = plsc.bitcast(_mono(k), jnp.float32)
    return tstar


def _build_block_lists(bm_ref, blk_l, thr):
    lane = _LANE()
    counts = []
    for c in range(NCHUNK):
        na_vec = jnp.zeros((16,), jnp.int32)
        for q in range(4):
            bmv = bm_ref[pl.ds((4 * c + q) * 16, 16)]
            amsk = bmv >= thr
            na = na_vec[0]
            plsc.store_compressed(blk_l.at[pl.ds(80 * c + na, 16)],
                                  q * 16 + lane, mask=amsk)
            na_vec = na_vec + plsc.all_reduce_population_count(amsk)
        counts.append(na_vec[0])
    return counts


def _scan_chunk(ck, c, na, blk_l, cand_v, cand_i, m_vec, thr):
    lane = _LANE()

    def blk(i, carry):
        m_vec, thr = carry
        blv = blk_l[pl.ds(80 * c + ((i >> 4) << 4), 16)]
        bid = lax.gather(
            blv, jnp.full((16, 1), i & 15, jnp.int32),
            lax.GatherDimensionNumbers(offset_dims=(),
                                       collapsed_slice_dims=(0,),
                                       start_index_map=(0,)),
            (1,), mode=lax.GatherScatterMode.PROMISE_IN_BOUNDS)
        base = bid[0] * BLK
        vs, msks, bases = [], [], []
        for j in range(8):
            v = ck[pl.ds(base + 16 * j, 16)]
            msk = v >= thr
            vs.append(v)
            msks.append(msk)
            bases.append(m_vec)
            m_vec = m_vec + plsc.all_reduce_population_count(msk)
        for j in range(8):
            m = bases[j][0]
            idx = (c * CH) + base + 16 * j + lane
            plsc.store_compressed(cand_v.at[pl.ds(m, 16)], vs[j],
                                  mask=msks[j])
            plsc.store_compressed(cand_i.at[pl.ds(m, 16)], idx,
                                  mask=msks[j])
        return (m_vec, thr)

    return lax.fori_loop(0, na, blk, (m_vec, thr))


def _maybe_compact(cand_v, cand_i, cand_k, m_vec, thr):
    m = m_vec[0]

    def do_compact(c2):
        m_vec2, thr2 = c2
        tstar = _compact64(cand_v, cand_i, cand_k, m_vec2[0])
        tv = plsc.bitcast(_mono(jnp.full((16,), tstar, jnp.int32)),
                          jnp.float32)
        return (jnp.full((16,), KTOP, jnp.int32), jnp.max(tv))

    return lax.cond(m > CAND - CH - 16, do_compact, lambda c2: c2,
                    (m_vec, thr))


def _extract64(cand_k, cand_i, outv, outi):
    lane = _LANE()

    def step(k, carry):
        wk, wi = carry
        vmax = jnp.full((16,), INT_MIN, jnp.int32)
        vcomb = jnp.full((16,), INT_MAX, jnp.int32)
        for j in range(KTOP // 16):
            cv = cand_k[pl.ds(16 * j, 16)]
            ci = cand_i[pl.ds(16 * j, 16)]
            comb = (ci << 8) | (16 * j + lane)
            better = (cv > vmax) | ((cv == vmax) & (comb < vcomb))
            vmax = jnp.where(better, cv, vmax)
            vcomb = jnp.where(better, comb, vcomb)
        mk = jnp.max(vmax)
        wc = jnp.min(jnp.where(vmax == mk, vcomb, INT_MAX))
        widx = wc >> 8
        wpos = wc & 255
        kmod = k & 15
        wk = jnp.where(lane == kmod, mk, wk)
        wi = jnp.where(lane == kmod, widx, wi)

        @pl.when(kmod == 15)
        def _():
            grp = k >> 4
            outv[pl.ds(16 * grp, 16)] = plsc.bitcast(_mono(wk), jnp.float32)
            outi[pl.ds(16 * grp, 16)] = wi

        wsrc = wpos >> 4
        cv = cand_k[pl.ds(16 * wsrc, 16)]
        cand_k[pl.ds(16 * wsrc, 16)] = jnp.where(
            lane == (wpos & 15), INT_MIN, cv)
        return (wk, wi)

    lax.fori_loop(0, KTOP, step,
                  (jnp.zeros((16,), jnp.int32), jnp.zeros((16,), jnp.int32)))


def _sc_topk(pre, bm, nrows):
    mesh = plsc.VectorSubcoreMesh(core_axis_name="c", subcore_axis_name="s")
    zeros16 = lambda: jnp.zeros((16,), jnp.float32)
    rpw = nrows // NW

    @functools.partial(
        pl.kernel,
        out_type=[jax.ShapeDtypeStruct((nrows, N_LAT), jnp.float32),
                  jax.ShapeDtypeStruct((nrows, KTOP), jnp.int32)],
        mesh=mesh,
        compiler_params=pltpu.CompilerParams(needs_layout_passes=False),
        scratch_types=[
            pltpu.VMEM((CH,), jnp.float32),
            pltpu.VMEM((CH,), jnp.float32),
            pltpu.VMEM((NBLK,), jnp.float32),
            pltpu.VMEM((NBLK,), jnp.float32),
            pltpu.VMEM((CAND + 16,), jnp.float32),
            pltpu.VMEM((CAND + 16,), jnp.int32),
            pltpu.VMEM((CAND + 16,), jnp.int32),
            pltpu.VMEM((336,), jnp.int32),
            pltpu.VMEM((N_LAT,), jnp.float32),
            pltpu.VMEM((N_LAT,), jnp.float32),
            pltpu.VMEM((KTOP,), jnp.float32),
            pltpu.VMEM((KTOP,), jnp.float32),
            pltpu.VMEM((KTOP,), jnp.int32),
            pltpu.VMEM((KTOP,), jnp.int32),
            pltpu.SemaphoreType.DMA,
            pltpu.SemaphoreType.DMA,
            pltpu.SemaphoreType.DMA,
            pltpu.SemaphoreType.DMA,
            pltpu.SemaphoreType.DMA,
            pltpu.SemaphoreType.DMA,
        ],
    )
    def k(pre_hbm, bm_hbm, acts_hbm, idx_hbm,
          ck_a, ck_b, bm_a, bm_b, cand_v, cand_i, cand_k, blk_l,
          acts_a, acts_b, outv_a, outv_b, outi_a, outi_b,
          sem_ca, sem_cb, sem_bma, sem_bmb, sem_oa, sem_ob):
        wid = lax.axis_index("s") * 2 + lax.axis_index("c")
        base = wid * rpw

        def z(j, _):
            acts_a[pl.ds(16 * j, 16)] = zeros16()
            acts_b[pl.ds(16 * j, 16)] = zeros16()
            return 0
        lax.fori_loop(0, N_LAT // 16, z, 0)

        pltpu.async_copy(bm_hbm.at[base], bm_a, sem_bma)

        def do_row(i, bm_mine, sem_bm_mine, bm_next, sem_bm_next,
                   acts_buf, outv, outi, sem_o):
            r = base + i
            nxt = base + jnp.minimum(i + 1, rpw - 1)
            pltpu.async_copy(bm_hbm.at[nxt], bm_next, sem_bm_next)
            h0 = pltpu.async_copy(pre_hbm.at[r, pl.ds(0, CH)], ck_a, sem_ca)
            h1 = pltpu.async_copy(pre_hbm.at[r, pl.ds(CH, CH)], ck_b, sem_cb)
            pltpu.make_async_copy(bm_hbm.at[r], bm_mine, sem_bm_mine).wait()
            thr = _bisect_t0(bm_mine)
            na = _build_block_lists(bm_mine, blk_l, thr)

            @pl.when(i >= 2)
            def _():
                pltpu.make_async_copy(acts_buf, acts_hbm.at[r], sem_o).wait()
                pltpu.make_async_copy(outi, idx_hbm.at[r], sem_o).wait()
                for g in range(KTOP // 16):
                    iv = outi[pl.ds(16 * g, 16)]
                    plsc.store_scatter(acts_buf, [iv], zeros16())

            mv = jnp.zeros((16,), jnp.int32)
            h0.wait()
            mv, thr = _scan_chunk(ck_a, 0, na[0], blk_l, cand_v, cand_i,
                                  mv, thr)
            h2 = pltpu.async_copy(pre_hbm.at[r, pl.ds(2 * CH, CH)], ck_a,
                                  sem_ca)
            h1.wait()
            mv, thr = _maybe_compact(cand_v, cand_i, cand_k, mv, thr)
            mv, thr = _scan_chunk(ck_b, 1, na[1], blk_l, cand_v, cand_i,
                                  mv, thr)
            h3 = pltpu.async_copy(pre_hbm.at[r, pl.ds(3 * CH, CH)], ck_b,
                                  sem_cb)
            h2.wait()
            mv, thr = _maybe_compact(cand_v, cand_i, cand_k, mv, thr)
            mv, thr = _scan_chunk(ck_a, 2, na[2], blk_l, cand_v, cand_i,
                                  mv, thr)
            h3.wait()
            mv, thr = _maybe_compact(cand_v, cand_i, cand_k, mv, thr)
            mv, thr = _scan_chunk(ck_b, 3, na[3], blk_l, cand_v, cand_i,
                                  mv, thr)

            _compact64(cand_v, cand_i, cand_k, mv[0])
            _extract64(cand_k, cand_i, outv, outi)

            for g in range(KTOP // 16):
                iv = outi[pl.ds(16 * g, 16)]
                vv = jnp.maximum(outv[pl.ds(16 * g, 16)], jnp.float32(0.0))
                plsc.store_scatter(acts_buf, [iv], vv)
            pltpu.async_copy(acts_buf, acts_hbm.at[r], sem_o)
            pltpu.async_copy(outi, idx_hbm.at[r], sem_o)

        def pair(p, _):
            do_row(2 * p, bm_a, sem_bma, bm_b, sem_bmb,
                   acts_a, outv_a, outi_a, sem_oa)
            do_row(2 * p + 1, bm_b, sem_bmb, bm_a, sem_bma,
                   acts_b, outv_b, outi_b, sem_ob)
            return 0
        lax.fori_loop(0, rpw // 2, pair, 0)

        pltpu.make_async_copy(bm_hbm.at[base], bm_a, sem_bma).wait()
        pltpu.make_async_copy(acts_a, acts_hbm.at[base], sem_oa).wait()
        pltpu.make_async_copy(outi_a, idx_hbm.at[base], sem_oa).wait()
        pltpu.make_async_copy(acts_b, acts_hbm.at[base], sem_ob).wait()
        pltpu.make_async_copy(outi_b, idx_hbm.at[base], sem_ob).wait()

    return k(pre, bm)


def kernel(x, W_enc, b_enc, W_dec, b_dec):
    Wd16 = W_dec.astype(jnp.bfloat16)
    NSPLIT = 4
    H = BATCH // NSPLIT
    recons, actss, idxs = [], [], []
    for h in range(NSPLIT):
        xh = lax.slice_in_dim(x, h * H, (h + 1) * H, axis=0)
        pre, bm = _encode(xh, W_enc, b_enc, H)
        acts_h, idx_h = _sc_topk(pre, bm, H)
        recons.append(_decode(acts_h, Wd16, b_dec, H))
        actss.append(acts_h)
        idxs.append(idx_h)
    recon = jnp.concatenate(recons, axis=0)
    acts = jnp.concatenate(actss, axis=0)
    topk_idx = jnp.concatenate(idxs, axis=0)
    return (recon, acts, topk_idx)


# speedup vs baseline: 1.1851x; 1.0243x over previous
import functools
import jax
import jax.numpy as jnp
from jax import lax
from jax.experimental import pallas as pl
from jax.experimental.pallas import tpu as pltpu
from jax.experimental.pallas import tpu_sc as plsc

D_IN = 2048
N_LAT = 32768
KTOP = 64
BATCH = 4096

BLK = 128
NBLK = N_LAT // BLK
NW = 32
ROWS_PER_W = BATCH // NW
NCHUNK = 4
CH = N_LAT // NCHUNK
CAND = 10496
INT_MIN = -2147483648
INT_MAX = 2147483647


def _enc_body(x_ref, w_ref, b_ref, out_ref, bm_ref):
    acc = lax.dot_general(x_ref[...], w_ref[...], (((1,), (1,)), ((), ())),
                          preferred_element_type=jnp.float32)
    acc = acc + b_ref[...]
    out_ref[...] = acc
    nb = acc.shape[1] // BLK
    bms = [jnp.max(acc[:, g * BLK:(g + 1) * BLK], axis=1, keepdims=True)
           for g in range(nb)]
    bm_ref[...] = jnp.concatenate(bms, axis=1)[None]


def _encode(x, W_enc, b_enc, nrows, BR=1024, BL=1024):
    grid = (nrows // BR, N_LAT // BL)
    nb = BL // BLK
    pre, bm3 = pl.pallas_call(
        _enc_body,
        grid=grid,
        in_specs=[
            pl.BlockSpec((BR, D_IN), lambda r, l: (r, 0)),
            pl.BlockSpec((BL, D_IN), lambda r, l: (l, 0)),
            pl.BlockSpec((1, BL), lambda r, l: (0, l)),
        ],
        out_specs=[
            pl.BlockSpec((BR, BL), lambda r, l: (r, l)),
            pl.BlockSpec((1, BR, nb), lambda r, l: (l, r, 0)),
        ],
        out_shape=[
            jax.ShapeDtypeStruct((nrows, N_LAT), jnp.float32),
            jax.ShapeDtypeStruct((N_LAT // BL, nrows, nb), jnp.float32),
        ],
    )(x, W_enc, b_enc.reshape(1, N_LAT))
    bm = bm3.transpose(1, 0, 2).reshape(nrows, NBLK)
    return pre, bm


def _dec_body(a_ref, w_ref, b_ref, out_ref):
    k = pl.program_id(1)
    a16 = a_ref[...].astype(jnp.bfloat16)
    acc = lax.dot_general(a16, w_ref[...], (((1,), (1,)), ((), ())),
                          preferred_element_type=jnp.float32)

    @pl.when(k == 0)
    def _():
        out_ref[...] = acc + b_ref[...]

    @pl.when(k > 0)
    def _():
        out_ref[...] = out_ref[...] + acc


def _decode(acts, Wd16, b_dec, nrows, BR=1024, BK=2048):
    grid = (nrows // BR, N_LAT // BK)
    return pl.pallas_call(
        _dec_body,
        grid=grid,
        in_specs=[
            pl.BlockSpec((BR, BK), lambda r, k: (r, k)),
            pl.BlockSpec((D_IN, BK), lambda r, k: (0, k)),
            pl.BlockSpec((1, D_IN), lambda r, k: (0, 0)),
        ],
        out_specs=pl.BlockSpec((BR, D_IN), lambda r, k: (r, 0)),
        out_shape=jax.ShapeDtypeStruct((nrows, D_IN), jnp.float32),
    )(acts, Wd16, b_dec.reshape(1, D_IN))


_LANE = lambda: lax.iota(jnp.int32, 16)


def _mono(b):
    return b ^ jnp.where(b < 0, jnp.int32(0x7FFFFFFF), jnp.int32(0))


def _count_ge_f(ref, nv16, thr):
    cnt = jnp.zeros((16,), jnp.int32)
    for j in range(nv16):
        v = ref[pl.ds(16 * j, 16)]
        cnt = cnt + jnp.where(v >= thr, 1, 0).astype(jnp.int32)
    return jnp.sum(cnt)


def _count_ge_k(kref, nv, thr):
    def body(j, cnt):
        v = kref[pl.ds(16 * j, 16)]
        return cnt + jnp.where(v >= thr, 1, 0).astype(jnp.int32)
    cnt = lax.fori_loop(0, nv, body, jnp.zeros((16,), jnp.int32))
    return jnp.sum(cnt)


def _bisect_t0(bm_ref):
    v0 = bm_ref[pl.ds(0, 16)]
    mx, mn = v0, v0
    for j in range(1, NBLK // 16):
        v = bm_ref[pl.ds(16 * j, 16)]
        mx = jnp.maximum(mx, v)
        mn = jnp.minimum(mn, v)
    hi = jnp.max(mx)
    lo = jnp.min(mn)

    def body(_, c):
        lo, hi = c
        mid = lo + (hi - lo) * jnp.float32(0.5)
        ge = _count_ge_f(bm_ref, NBLK // 16, mid) >= KTOP
        return (jnp.where(ge, mid, lo), jnp.where(ge, hi, mid))

    lo, hi = lax.fori_loop(0, 18, body, (lo, hi))
    return lo


def _compact64(cand_v, cand_i, cand_k, m):
    nv = (m + 15) >> 4

    def monoify(j, _):
        b = plsc.bitcast(cand_v[pl.ds(16 * j, 16)], jnp.int32)
        cand_k[pl.ds(16 * j, 16)] = _mono(b)
        return 0
    lax.fori_loop(0, nv, monoify, 0)
    sent = jnp.full((16,), INT_MIN, jnp.int32)
    cand_k[pl.ds(m, 16)] = sent

    def bis_static(_, c):
        lo, hi = c
        mid = (lo >> 1) + (hi >> 1) + (lo & hi & 1)
        cnt = jnp.zeros((16,), jnp.int32)
        for j in range(16):
            v = cand_k[pl.ds(16 * j, 16)]
            cnt = cnt + jnp.where(v >= mid, 1, 0).astype(jnp.int32)
        ge = jnp.sum(cnt) >= KTOP
        return (jnp.where(ge, mid, lo), jnp.where(ge, hi, mid))

    def bis_dyn(_, c):
        lo, hi = c
        mid = (lo >> 1) + (hi >> 1) + (lo & hi & 1)
        ge = _count_ge_k(cand_k, nv, mid) >= KTOP
        return (jnp.where(ge, mid, lo), jnp.where(ge, hi, mid))

    def run_static(_):
        def pad(j, _):
            cand_k[pl.ds(16 * j, 16)] = sent
            return 0
        lax.fori_loop((m + 16) >> 4, 16, pad, 0)
        t, _ = lax.fori_loop(0, 32, bis_static,
                             (jnp.int32(INT_MIN), jnp.int32(INT_MAX)))
        return t

    def run_dyn(_):
        t, _ = lax.fori_loop(0, 32, bis_dyn,
                             (jnp.int32(INT_MIN), jnp.int32(INT_MAX)))
        return t

    tstar = lax.cond(m <= 240, run_static, run_dyn, 0)

    def cgt(j, cnt):
        v = cand_k[pl.ds(16 * j, 16)]
        return cnt + jnp.sum(jnp.where(v > tstar, 1, 0).astype(jnp.int32))
    count_gt = lax.fori_loop(0, nv, cgt, jnp.int32(0))
    need_eq = KTOP - count_gt

    def cpart(j, carry):
        mo, ke = carry
        v = cand_k[pl.ds(16 * j, 16)]
        ci = cand_i[pl.ds(16 * j, 16)]
        gt = v > tstar
        eq = v == tstar
        eq_rank = ke + jnp.cumsum(eq.astype(jnp.int32))
        keep = gt | (eq & (eq_rank <= need_eq))
        plsc.store_compressed(cand_k.at[pl.ds(mo, 16)], v, mask=keep)
        plsc.store_compressed(cand_i.at[pl.ds(mo, 16)], ci, mask=keep)
        return (mo + jnp.sum(keep.astype(jnp.int32)),
                ke + jnp.sum(eq.astype(jnp.int32)))
    lax.fori_loop(0, nv, cpart, (jnp.int32(0), jnp.int32(0)))

    for g in range(KTOP // 16):
        k = cand_k[pl.ds(16 * g, 16)]
        cand_v[pl.ds(16 * g, 16)] = plsc.bitcast(_mono(k), jnp.float32)
    return tstar


def _build_block_lists(bm_ref, blk_l, thr):
    lane = _LANE()
    counts = []
    for c in range(NCHUNK):
        na_vec = jnp.zeros((16,), jnp.int32)
        for q in range(4):
            bmv = bm_ref[pl.ds((4 * c + q) * 16, 16)]
            amsk = bmv >= thr
            na = na_vec[0]
            plsc.store_compressed(blk_l.at[pl.ds(80 * c + na, 16)],
                                  q * 16 + lane, mask=amsk)
            na_vec = na_vec + plsc.all_reduce_population_count(amsk)
        counts.append(na_vec[0])
    return counts


def _scan_chunk(ck, c, na, blk_l, cand_v, cand_i, m_vec, thr):
    lane = _LANE()

    def blk(i, carry):
        m_vec, thr = carry
        blv = blk_l[pl.ds(80 * c + ((i >> 4) << 4), 16)]
        bid = lax.gather(
            blv, jnp.full((16, 1), i & 15, jnp.int32),
            lax.GatherDimensionNumbers(offset_dims=(),
                                       collapsed_slice_dims=(0,),
                                       start_index_map=(0,)),
            (1,), mode=lax.GatherScatterMode.PROMISE_IN_BOUNDS)
        base = bid[0] * BLK
        vs, msks, bases = [], [], []
        for j in range(8):
            v = ck[pl.ds(base + 16 * j, 16)]
            msk = v >= thr
            vs.append(v)
            msks.append(msk)
            bases.append(m_vec)
            m_vec = m_vec + plsc.all_reduce_population_count(msk)
        for j in range(8):
            m = bases[j][0]
            idx = (c * CH) + base + 16 * j + lane
            plsc.store_compressed(cand_v.at[pl.ds(m, 16)], vs[j],
                                  mask=msks[j])
            plsc.store_compressed(cand_i.at[pl.ds(m, 16)], idx,
                                  mask=msks[j])
        return (m_vec, thr)

    return lax.fori_loop(0, na, blk, (m_vec, thr))


def _maybe_compact(cand_v, cand_i, cand_k, m_vec, thr):
    m = m_vec[0]

    def do_compact(c2):
        m_vec2, thr2 = c2
        tstar = _compact64(cand_v, cand_i, cand_k, m_vec2[0])
        tv = plsc.bitcast(_mono(jnp.full((16,), tstar, jnp.int32)),
                          jnp.float32)
        return (jnp.full((16,), KTOP, jnp.int32), jnp.max(tv))

    return lax.cond(m > CAND - CH - 16, do_compact, lambda c2: c2,
                    (m_vec, thr))


def _extract64(cand_k, cand_i, outv, outi):
    lane = _LANE()
    nv = KTOP // 16
    ks = [cand_k[pl.ds(16 * j, 16)] for j in range(nv)]
    cs = [(cand_i[pl.ds(16 * j, 16)] << 8) | (16 * j + lane)
          for j in range(nv)]

    def step(k, carry):
        wk, wi, ks, cs = carry[0], carry[1], list(carry[2]), list(carry[3])
        vmax = jnp.full((16,), INT_MIN, jnp.int32)
        vcomb = jnp.full((16,), INT_MAX, jnp.int32)
        for j in range(nv):
            better = (ks[j] > vmax) | ((ks[j] == vmax) & (cs[j] < vcomb))
            vmax = jnp.where(better, ks[j], vmax)
            vcomb = jnp.where(better, cs[j], vcomb)
        mk = jnp.max(vmax)
        wc = jnp.min(jnp.where(vmax == mk, vcomb, INT_MAX))
        kmod = k & 15
        wk = jnp.where(lane == kmod, mk, wk)
        wi = jnp.where(lane == kmod, wc >> 8, wi)

        @pl.when(kmod == 15)
        def _():
            grp = k >> 4
            outv[pl.ds(16 * grp, 16)] = plsc.bitcast(_mono(wk), jnp.float32)
            outi[pl.ds(16 * grp, 16)] = wi

        ks = [jnp.where(cs[j] == wc, INT_MIN, ks[j]) for j in range(nv)]
        return (wk, wi, tuple(ks), tuple(cs))

    lax.fori_loop(0, KTOP, step,
                  (jnp.zeros((16,), jnp.int32), jnp.zeros((16,), jnp.int32),
                   tuple(ks), tuple(cs)))


def _sc_topk(pre, bm, nrows):
    mesh = plsc.VectorSubcoreMesh(core_axis_name="c", subcore_axis_name="s")
    zeros16 = lambda: jnp.zeros((16,), jnp.float32)
    rpw = nrows // NW

    @functools.partial(
        pl.kernel,
        out_type=[jax.ShapeDtypeStruct((nrows, N_LAT), jnp.float32),
                  jax.ShapeDtypeStruct((nrows, KTOP), jnp.int32)],
        mesh=mesh,
        compiler_params=pltpu.CompilerParams(needs_layout_passes=False),
        scratch_types=[
            pltpu.VMEM((CH,), jnp.float32),
            pltpu.VMEM((CH,), jnp.float32),
            pltpu.VMEM((NBLK,), jnp.float32),
            pltpu.VMEM((NBLK,), jnp.float32),
            pltpu.VMEM((CAND + 16,), jnp.float32),
            pltpu.VMEM((CAND + 16,), jnp.int32),
            pltpu.VMEM((CAND + 16,), jnp.int32),
            pltpu.VMEM((336,), jnp.int32),
            pltpu.VMEM((N_LAT,), jnp.float32),
            pltpu.VMEM((N_LAT,), jnp.float32),
            pltpu.VMEM((KTOP,), jnp.float32),
            pltpu.VMEM((KTOP,), jnp.float32),
            pltpu.VMEM((KTOP,), jnp.int32),
            pltpu.VMEM((KTOP,), jnp.int32),
            pltpu.SemaphoreType.DMA,
            pltpu.SemaphoreType.DMA,
            pltpu.SemaphoreType.DMA,
            pltpu.SemaphoreType.DMA,
            pltpu.SemaphoreType.DMA,
            pltpu.SemaphoreType.DMA,
        ],
    )
    def k(pre_hbm, bm_hbm, acts_hbm, idx_hbm,
          ck_a, ck_b, bm_a, bm_b, cand_v, cand_i, cand_k, blk_l,
          acts_a, acts_b, outv_a, outv_b, outi_a, outi_b,
          sem_ca, sem_cb, sem_bma, sem_bmb, sem_oa, sem_ob):
        wid = lax.axis_index("s") * 2 + lax.axis_index("c")
        base = wid * rpw

        def z(j, _):
            acts_a[pl.ds(16 * j, 16)] = zeros16()
            acts_b[pl.ds(16 * j, 16)] = zeros16()
            return 0
        lax.fori_loop(0, N_LAT // 16, z, 0)

        pltpu.async_copy(bm_hbm.at[base], bm_a, sem_bma)

        def do_row(i, bm_mine, sem_bm_mine, bm_next, sem_bm_next,
                   acts_buf, outv, outi, sem_o):
            r = base + i
            nxt = base + jnp.minimum(i + 1, rpw - 1)
            pltpu.async_copy(bm_hbm.at[nxt], bm_next, sem_bm_next)
            h0 = pltpu.async_copy(pre_hbm.at[r, pl.ds(0, CH)], ck_a, sem_ca)
            h1 = pltpu.async_copy(pre_hbm.at[r, pl.ds(CH, CH)], ck_b, sem_cb)
            pltpu.make_async_copy(bm_hbm.at[r], bm_mine, sem_bm_mine).wait()
            thr = _bisect_t0(bm_mine)
            na = _build_block_lists(bm_mine, blk_l, thr)

            @pl.when(i >= 2)
            def _():
                pltpu.make_async_copy(acts_buf, acts_hbm.at[r], sem_o).wait()
                pltpu.make_async_copy(outi, idx_hbm.at[r], sem_o).wait()
                for g in range(KTOP // 16):
                    iv = outi[pl.ds(16 * g, 16)]
                    plsc.store_scatter(acts_buf, [iv], zeros16())

            mv = jnp.zeros((16,), jnp.int32)
            h0.wait()
            mv, thr = _scan_chunk(ck_a, 0, na[0], blk_l, cand_v, cand_i,
                                  mv, thr)
            h2 = pltpu.async_copy(pre_hbm.at[r, pl.ds(2 * CH, CH)], ck_a,
                                  sem_ca)
            h1.wait()
            mv, thr = _maybe_compact(cand_v, cand_i, cand_k, mv, thr)
            mv, thr = _scan_chunk(ck_b, 1, na[1], blk_l, cand_v, cand_i,
                                  mv, thr)
            h3 = pltpu.async_copy(pre_hbm.at[r, pl.ds(3 * CH, CH)], ck_b,
                                  sem_cb)
            h2.wait()
            mv, thr = _maybe_compact(cand_v, cand_i, cand_k, mv, thr)
            mv, thr = _scan_chunk(ck_a, 2, na[2], blk_l, cand_v, cand_i,
                                  mv, thr)
            h3.wait()
            mv, thr = _maybe_compact(cand_v, cand_i, cand_k, mv, thr)
            mv, thr = _scan_chunk(ck_b, 3, na[3], blk_l, cand_v, cand_i,
                                  mv, thr)

            _compact64(cand_v, cand_i, cand_k, mv[0])
            _extract64(cand_k, cand_i, outv, outi)

            for g in range(KTOP // 16):
                iv = outi[pl.ds(16 * g, 16)]
                vv = jnp.maximum(outv[pl.ds(16 * g, 16)], jnp.float32(0.0))
                plsc.store_scatter(acts_buf, [iv], vv)
            pltpu.async_copy(acts_buf, acts_hbm.at[r], sem_o)
            pltpu.async_copy(outi, idx_hbm.at[r], sem_o)

        def pair(p, _):
            do_row(2 * p, bm_a, sem_bma, bm_b, sem_bmb,
                   acts_a, outv_a, outi_a, sem_oa)
            do_row(2 * p + 1, bm_b, sem_bmb, bm_a, sem_bma,
                   acts_b, outv_b, outi_b, sem_ob)
            return 0
        lax.fori_loop(0, rpw // 2, pair, 0)

        pltpu.make_async_copy(bm_hbm.at[base], bm_a, sem_bma).wait()
        pltpu.make_async_copy(acts_a, acts_hbm.at[base], sem_oa).wait()
        pltpu.make_async_copy(outi_a, idx_hbm.at[base], sem_oa).wait()
        pltpu.make_async_copy(acts_b, acts_hbm.at[base], sem_ob).wait()
        pltpu.make_async_copy(outi_b, idx_hbm.at[base], sem_ob).wait()

    return k(pre, bm)


def kernel(x, W_enc, b_enc, W_dec, b_dec):
    Wd16 = W_dec.astype(jnp.bfloat16)
    NSPLIT = 4
    H = BATCH // NSPLIT
    recons, actss, idxs = [], [], []
    for h in range(NSPLIT):
        xh = lax.slice_in_dim(x, h * H, (h + 1) * H, axis=0)
        pre, bm = _encode(xh, W_enc, b_enc, H)
        acts_h, idx_h = _sc_topk(pre, bm, H)
        recons.append(_decode(acts_h, Wd16, b_dec, H))
        actss.append(acts_h)
        idxs.append(idx_h)
    recon = jnp.concatenate(recons, axis=0)
    acts = jnp.concatenate(actss, axis=0)
    topk_idx = jnp.concatenate(idxs, axis=0)
    return (recon, acts, topk_idx)
